# Initial kernel scaffold; baseline (speedup 1.0000x reference)
import functools

import jax
import jax.numpy as jnp
from jax.experimental import pallas as pl
from jax.experimental.pallas import tpu as pltpu
from jax.experimental.pallas import tpu_sc as plsc

_N = 50000
_K = 16
_E = _N * _K
_T = 200
_GW = 1000


def _sc_gather(tbl, cols):
    e = cols.shape[0]
    width = tbl.shape[1]
    mesh = plsc.VectorSubcoreMesh(core_axis_name="core", subcore_axis_name="subcore")
    idx = cols.reshape(1, e)

    @functools.partial(
        pl.kernel,
        out_type=jax.ShapeDtypeStruct((e, width), tbl.dtype),
        mesh=mesh,
    )
    def gather_kernel(x_hbm, i_hbm, o_hbm):
        def body(i_vmem, o_vmem):
            pltpu.sync_copy(x_hbm.at[i_vmem.at[0]], o_vmem)

        pltpu.emit_pipeline(
            body,
            grid=(e // _GW,),
            in_specs=[pl.BlockSpec((1, _GW), lambda i: (0, i))],
            out_specs=[pl.BlockSpec((_GW, width), lambda i: (i, 0))],
            core_axis_name=("core", "subcore"),
            dimension_semantics=(pltpu.PARALLEL,),
        )(i_hbm, o_hbm)

    return gather_kernel(tbl, idx)


def _mlp(x, w1, b1, w2, b2):
    h = jnp.maximum(jnp.dot(x, w1[...], preferred_element_type=jnp.float32) + b1[...], 0.0)
    return jnp.dot(h, w2[...], preferred_element_type=jnp.float32) + b2[...]


def _norm3(v):
    return jnp.sqrt(jnp.sum(v * v, axis=1, keepdims=True))


def _cross3(a, b):
    a0, a1, a2 = a[:, 0:1], a[:, 1:2], a[:, 2:3]
    b0, b1, b2 = b[:, 0:1], b[:, 1:2], b[:, 2:3]
    return jnp.concatenate(
        [a1 * b2 - a2 * b1, a2 * b0 - a0 * b2, a0 * b1 - a1 * b0], axis=1
    )


def _angle(v1, v2):
    c = _cross3(v1, v2)
    dot = jnp.sum(v1 * v2, axis=1, keepdims=True)
    return jnp.arctan2(_norm3(c), dot)


def _tc_body(std_ref, tbl_ref, g_ref, w_ref, *rest):
    (l1w1, l1b1, l1w2, l1b2, gw1, gb1, gw2, gb2,
     l2w1, l2b1, l2w2, l2b2, g2w1, g2b1, g2w2, g2b2,
     l3w1, l3b1, l3w2, l3b2, g3w1, g3b1, g3w2, g3b2,
     l4w1, l4b1, l4w2, l4b2, out_ref) = rest

    t = _T
    k = _K
    tk = t * k
    s = 0.2 / std_ref[0, 0]

    def bcast(x):
        f = x.shape[1]
        return jnp.broadcast_to(x[:, None, :], (t, k, f)).reshape(tk, f)

    tbl = tbl_ref[...]
    pr_n = tbl[:, 0:3]
    nr_n = tbl[:, 3:6]
    pr = bcast(pr_n)
    nr = bcast(nr_n)
    g = g_ref[...]
    pc = g[:, 0:3]
    nc = g[:, 3:6]

    cart = (pc - pr) * s
    ppf = jnp.concatenate(
        [_norm3(cart), _angle(nr, cart), _angle(nc, cart), _angle(nr, nc)], axis=1
    )
    x = jnp.concatenate([cart, w_ref[...], ppf], axis=1)
    x = _mlp(x, l1w1, l1b1, l1w2, l1b2)

    gx = jnp.mean(x.reshape(t, k, 16), axis=1)
    xg = _mlp(jnp.concatenate([gx, nr_n], axis=1), gw1, gb1, gw2, gb2)
    x = _mlp(jnp.concatenate([x, bcast(xg)], axis=1), l2w1, l2b1, l2w2, l2b2)

    gx = jnp.mean(x.reshape(t, k, 16), axis=1)
    xg = _mlp(gx, g2w1, g2b1, g2w2, g2b2)
    x = _mlp(jnp.concatenate([x, bcast(xg)], axis=1), l3w1, l3b1, l3w2, l3b2)

    gx = jnp.mean(x.reshape(t, k, 16), axis=1)
    xg = _mlp(gx, g3w1, g3b1, g3w2, g3b2)

    quat = xg[:, 0:4]
    qn = quat / (jnp.sqrt(jnp.sum(quat * quat, axis=1, keepdims=True)) + 1e-8)
    qw, qx, qy, qz = qn[:, 0:1], qn[:, 1:2], qn[:, 2:3], qn[:, 3:4]
    m = jnp.concatenate(
        [
            1 - 2 * (qy * qy + qz * qz), 2 * (qx * qy - qw * qz), 2 * (qx * qz + qw * qy),
            2 * (qx * qy + qw * qz), 1 - 2 * (qx * qx + qz * qz), 2 * (qy * qz - qw * qx),
            2 * (qx * qz - qw * qy), 2 * (qy * qz + qw * qx), 1 - 2 * (qx * qx + qy * qy),
        ],
        axis=1,
    )
    me = bcast(m)
    c0, c1, c2 = cart[:, 0:1], cart[:, 1:2], cart[:, 2:3]
    rc = jnp.concatenate(
        [
            me[:, 0:1] * c0 + me[:, 1:2] * c1 + me[:, 2:3] * c2,
            me[:, 3:4] * c0 + me[:, 4:5] * c1 + me[:, 5:6] * c2,
            me[:, 6:7] * c0 + me[:, 7:8] * c1 + me[:, 8:9] * c2,
        ],
        axis=1,
    )

    x = jnp.concatenate([x, bcast(xg[:, 4:12]), rc], axis=1)
    y = _mlp(x, l4w1, l4b1, l4w2, l4b2)

    y3 = y.reshape(t, k, 1)
    ymax = jnp.max(y3, axis=1, keepdims=True)
    ey = jnp.exp(y3 - ymax)
    out_ref[...] = ey / jnp.sum(ey, axis=1, keepdims=True)


def _tc_forward(std, tbl, g, w, weights):
    tk = _T * _K
    nblk = _N // _T
    in_specs = [
        pl.BlockSpec(memory_space=pltpu.SMEM),
        pl.BlockSpec((_T, 8), lambda i: (i, 0)),
        pl.BlockSpec((tk, 8), lambda i: (i, 0)),
        pl.BlockSpec((tk, 1), lambda i: (i, 0)),
    ] + [pl.BlockSpec(wa.shape, lambda i: tuple([0] * wa.ndim)) for wa in weights]
    out = pl.pallas_call(
        _tc_body,
        grid=(nblk,),
        in_specs=in_specs,
        out_specs=pl.BlockSpec((_T, _K, 1), lambda i: (i, 0, 0)),
        out_shape=jax.ShapeDtypeStruct((_N, _K, 1), jnp.float32),
    )(std, tbl, g, w, *weights)
    return out.reshape(_N, _K)


def kernel(pos, old_weights, normals, edge_index, dense_l, stddev, params):
    n = pos.shape[0]
    cols = edge_index[1]
    tbl = jnp.concatenate(
        [pos, normals, jnp.zeros((n, 2), pos.dtype)], axis=1
    )
    g = _sc_gather(tbl, cols)
    w = old_weights.reshape(-1, 1)
    std = stddev.reshape(1, 1)
    weights = []
    for name in ("layer1", "layerg", "layer2", "layerg2", "layer3", "layerg3", "layer4"):
        w1, b1, w2, b2 = params[name]
        weights += [w1, b1.reshape(1, -1), w2, b2.reshape(1, -1)]
    return _tc_forward(std, tbl, g, w, weights)


# speedup vs baseline: 2.2238x; 2.2238x over previous
import functools

import jax
import jax.numpy as jnp
from jax import lax
from jax.experimental import pallas as pl
from jax.experimental.pallas import tpu as pltpu
from jax.experimental.pallas import tpu_sc as plsc

_N = 50000
_K = 16
_E = _N * _K
_T = 200
_NC = 2
_NS = 16
_CH = 5000


def _sc_gather6(tables, cols):
    e = cols.shape[0]
    nw = _NC * _NS
    b_per_w = e // nw
    n_ch = b_per_w // _CH
    mesh = plsc.VectorSubcoreMesh(core_axis_name="c", subcore_axis_name="s")

    @functools.partial(
        pl.kernel,
        out_type=[jax.ShapeDtypeStruct((e,), jnp.float32) for _ in range(6)],
        mesh=mesh,
        scratch_types=[pltpu.VMEM((_CH,), jnp.int32)]
        + [pltpu.VMEM((_CH,), jnp.float32) for _ in range(6)]
        + [pltpu.SemaphoreType.DMA],
    )
    def gather_kernel(*refs):
        tbls = refs[0:6]
        idx_hbm = refs[6]
        outs = refs[7:13]
        idx_v = refs[13]
        vals = refs[14:20]
        sem = refs[20]
        wid = lax.axis_index("s") * _NC + lax.axis_index("c")
        base = wid * b_per_w

        @pl.loop(0, n_ch)
        def _(c):
            off = base + c * _CH
            pltpu.sync_copy(idx_hbm.at[pl.ds(off, _CH)], idx_v)
            copies = [
                pltpu.async_copy(tbls[j].at[idx_v], vals[j], sem)
                for j in range(6)
            ]
            for cp in copies:
                cp.wait()
            for j in range(6):
                pltpu.sync_copy(vals[j], outs[j].at[pl.ds(off, _CH)])

    return gather_kernel(*tables, cols)


def _mlp(x, w1, b1, w2, b2):
    h = jnp.maximum(jnp.dot(x, w1[...], preferred_element_type=jnp.float32) + b1[...], 0.0)
    return jnp.dot(h, w2[...], preferred_element_type=jnp.float32) + b2[...]


def _norm3(v):
    return jnp.sqrt(jnp.sum(v * v, axis=1, keepdims=True))


def _cross3(a, b):
    a0, a1, a2 = a[:, 0:1], a[:, 1:2], a[:, 2:3]
    b0, b1, b2 = b[:, 0:1], b[:, 1:2], b[:, 2:3]
    return jnp.concatenate(
        [a1 * b2 - a2 * b1, a2 * b0 - a0 * b2, a0 * b1 - a1 * b0], axis=1
    )


def _angle(v1, v2):
    c = _cross3(v1, v2)
    dot = jnp.sum(v1 * v2, axis=1, keepdims=True)
    return jnp.arctan2(_norm3(c), dot)


def _tc_body(std_ref, pos_ref, nrm_ref, g0, g1, g2, g3, g4, g5, w_ref, *rest):
    (l1w1, l1b1, l1w2, l1b2, gw1, gb1, gw2, gb2,
     l2w1, l2b1, l2w2, l2b2, g2w1, g2b1, g2w2, g2b2,
     l3w1, l3b1, l3w2, l3b2, g3w1, g3b1, g3w2, g3b2,
     l4w1, l4b1, l4w2, l4b2, out_ref) = rest

    t = _T
    k = _K
    tk = t * k
    s = 0.2 / std_ref[0, 0]

    def bcast(x):
        f = x.shape[1]
        return jnp.broadcast_to(x[:, None, :], (t, k, f)).reshape(tk, f)

    pr_n = pos_ref[...]
    nr_n = nrm_ref[...]
    pr = bcast(pr_n)
    nr = bcast(nr_n)
    pc = jnp.concatenate([g0[...], g1[...], g2[...]], axis=1)
    nc = jnp.concatenate([g3[...], g4[...], g5[...]], axis=1)

    cart = (pc - pr) * s
    ppf = jnp.concatenate(
        [_norm3(cart), _angle(nr, cart), _angle(nc, cart), _angle(nr, nc)], axis=1
    )
    x = jnp.concatenate([cart, w_ref[...], ppf], axis=1)
    x = _mlp(x, l1w1, l1b1, l1w2, l1b2)

    gx = jnp.mean(x.reshape(t, k, 16), axis=1)
    xg = _mlp(jnp.concatenate([gx, nr_n], axis=1), gw1, gb1, gw2, gb2)
    x = _mlp(jnp.concatenate([x, bcast(xg)], axis=1), l2w1, l2b1, l2w2, l2b2)

    gx = jnp.mean(x.reshape(t, k, 16), axis=1)
    xg = _mlp(gx, g2w1, g2b1, g2w2, g2b2)
    x = _mlp(jnp.concatenate([x, bcast(xg)], axis=1), l3w1, l3b1, l3w2, l3b2)

    gx = jnp.mean(x.reshape(t, k, 16), axis=1)
    xg = _mlp(gx, g3w1, g3b1, g3w2, g3b2)

    quat = xg[:, 0:4]
    qn = quat / (jnp.sqrt(jnp.sum(quat * quat, axis=1, keepdims=True)) + 1e-8)
    qw, qx, qy, qz = qn[:, 0:1], qn[:, 1:2], qn[:, 2:3], qn[:, 3:4]
    m = jnp.concatenate(
        [
            1 - 2 * (qy * qy + qz * qz), 2 * (qx * qy - qw * qz), 2 * (qx * qz + qw * qy),
            2 * (qx * qy + qw * qz), 1 - 2 * (qx * qx + qz * qz), 2 * (qy * qz - qw * qx),
            2 * (qx * qz - qw * qy), 2 * (qy * qz + qw * qx), 1 - 2 * (qx * qx + qy * qy),
        ],
        axis=1,
    )
    me = bcast(m)
    c0, c1, c2 = cart[:, 0:1], cart[:, 1:2], cart[:, 2:3]
    rc = jnp.concatenate(
        [
            me[:, 0:1] * c0 + me[:, 1:2] * c1 + me[:, 2:3] * c2,
            me[:, 3:4] * c0 + me[:, 4:5] * c1 + me[:, 5:6] * c2,
            me[:, 6:7] * c0 + me[:, 7:8] * c1 + me[:, 8:9] * c2,
        ],
        axis=1,
    )

    x = jnp.concatenate([x, bcast(xg[:, 4:12]), rc], axis=1)
    y = _mlp(x, l4w1, l4b1, l4w2, l4b2)

    y3 = y.reshape(t, k, 1)
    ymax = jnp.max(y3, axis=1, keepdims=True)
    ey = jnp.exp(y3 - ymax)
    out_ref[...] = ey / jnp.sum(ey, axis=1, keepdims=True)


def _tc_forward(std, pos, nrm, g6, w, weights):
    tk = _T * _K
    nblk = _N // _T
    in_specs = [
        pl.BlockSpec(memory_space=pltpu.SMEM),
        pl.BlockSpec((_T, 3), lambda i: (i, 0)),
        pl.BlockSpec((_T, 3), lambda i: (i, 0)),
    ] + [pl.BlockSpec((tk, 1), lambda i: (i, 0)) for _ in range(6)] + [
        pl.BlockSpec((tk, 1), lambda i: (i, 0)),
    ] + [pl.BlockSpec(wa.shape, lambda i: tuple([0] * wa.ndim)) for wa in weights]
    out = pl.pallas_call(
        _tc_body,
        grid=(nblk,),
        in_specs=in_specs,
        out_specs=pl.BlockSpec((_T, _K, 1), lambda i: (i, 0, 0)),
        out_shape=jax.ShapeDtypeStruct((_N, _K, 1), jnp.float32),
    )(std, pos, nrm, *g6, w, *weights)
    return out.reshape(_N, _K)


def kernel(pos, old_weights, normals, edge_index, dense_l, stddev, params):
    cols = edge_index[1]
    tables = [pos[:, 0], pos[:, 1], pos[:, 2],
              normals[:, 0], normals[:, 1], normals[:, 2]]
    g6 = [a.reshape(-1, 1) for a in _sc_gather6(tables, cols)]
    w = old_weights.reshape(-1, 1)
    std = stddev.reshape(1, 1)
    weights = []
    for name in ("layer1", "layerg", "layer2", "layerg2", "layer3", "layerg3", "layer4"):
        w1, b1, w2, b2 = params[name]
        weights += [w1, b1.reshape(1, -1), w2, b2.reshape(1, -1)]
    return _tc_forward(std, pos, normals, g6, w, weights)


# speedup vs baseline: 17.1995x; 7.7342x over previous
import functools

import jax
import jax.numpy as jnp
from jax import lax
from jax.experimental import pallas as pl
from jax.experimental.pallas import tpu as pltpu
from jax.experimental.pallas import tpu_sc as plsc

_N = 50000
_K = 16
_E = _N * _K
_T = 400
_NC = 2
_NS = 16
_CH = 5000


def _sc_gather6(tables, cols):
    e = cols.shape[0]
    nw = _NC * _NS
    b_per_w = e // nw
    n_ch = b_per_w // _CH
    mesh = plsc.VectorSubcoreMesh(core_axis_name="c", subcore_axis_name="s")

    @functools.partial(
        pl.kernel,
        out_type=[jax.ShapeDtypeStruct((e,), jnp.float32) for _ in range(6)],
        mesh=mesh,
        scratch_types=[pltpu.VMEM((_CH,), jnp.int32)]
        + [pltpu.VMEM((_CH,), jnp.float32) for _ in range(6)]
        + [pltpu.SemaphoreType.DMA],
    )
    def gather_kernel(*refs):
        tbls = refs[0:6]
        idx_hbm = refs[6]
        outs = refs[7:13]
        idx_v = refs[13]
        vals = refs[14:20]
        sem = refs[20]
        wid = lax.axis_index("s") * _NC + lax.axis_index("c")
        base = wid * b_per_w

        @pl.loop(0, n_ch)
        def _(c):
            off = base + c * _CH
            pltpu.sync_copy(idx_hbm.at[pl.ds(off, _CH)], idx_v)
            copies = [
                pltpu.async_copy(tbls[j].at[idx_v], vals[j], sem)
                for j in range(6)
            ]
            for cp in copies:
                cp.wait()
            for j in range(6):
                pltpu.sync_copy(vals[j], outs[j].at[pl.ds(off, _CH)])

    return gather_kernel(*tables, cols)


def _angle16(c0, c1, c2, d):
    return jnp.arctan2(jnp.sqrt(c0 * c0 + c1 * c1 + c2 * c2), d)


def _tc_body(std_ref, pos_ref, nrm_ref, g0, g1, g2, g3, g4, g5, w_ref, *rest):
    (rmat,
     p1, b1t, bd12, b2t1,
     gw1, gb1, gw2, gb2,
     bda2, tb2, b1t2, bd22, b2t2,
     g2w1, g2b1, g2w2, g2b2,
     bda3, tb3, b1t3, bd23, b2t3,
     g3w1, g3b1, g3w2, g3b2,
     bd4a, tb4, pc4, b4t, bd4b, b4b,
     out_ref) = rest

    def mm(a, b):
        return jnp.dot(a, b[...], preferred_element_type=jnp.float32)

    s = 0.2 / std_ref[0, 0]
    prx, pry, prz = pos_ref[:, 0:1], pos_ref[:, 1:2], pos_ref[:, 2:3]
    nrx, nry, nrz = nrm_ref[:, 0:1], nrm_ref[:, 1:2], nrm_ref[:, 2:3]
    pcx, pcy, pcz = g0[...], g1[...], g2[...]
    ncx, ncy, ncz = g3[...], g4[...], g5[...]

    cx = (pcx - prx) * s
    cy = (pcy - pry) * s
    cz = (pcz - prz) * s
    cn = jnp.sqrt(cx * cx + cy * cy + cz * cz)
    a1 = _angle16(nry * cz - nrz * cy, nrz * cx - nrx * cz, nrx * cy - nry * cx,
                  nrx * cx + nry * cy + nrz * cz)
    a2 = _angle16(ncy * cz - ncz * cy, ncz * cx - ncx * cz, ncx * cy - ncy * cx,
                  ncx * cx + ncy * cy + ncz * cz)
    a3 = _angle16(nry * ncz - nrz * ncy, nrz * ncx - nrx * ncz,
                  nrx * ncy - nry * ncx,
                  nrx * ncx + nry * ncy + nrz * ncz)

    x128 = jnp.concatenate([cx, cy, cz, w_ref[...], cn, a1, a2, a3], axis=1)
    h = jnp.maximum(mm(x128, p1) + b1t[...], 0.0)
    x16 = mm(h, bd12) + b2t1[...]

    gx = mm(x16, rmat)
    gin = jnp.concatenate([gx, nrm_ref[...]], axis=1)
    hg = jnp.maximum(mm(gin, gw1) + gb1[...], 0.0)
    xg = mm(hg, gw2) + gb2[...]

    h = jnp.maximum(mm(x16, bda2) + mm(xg, tb2) + b1t2[...], 0.0)
    x16 = mm(h, bd22) + b2t2[...]

    gx = mm(x16, rmat)
    hg = jnp.maximum(mm(gx, g2w1) + g2b1[...], 0.0)
    xg = mm(hg, g2w2) + g2b2[...]

    h = jnp.maximum(mm(x16, bda3) + mm(xg, tb3) + b1t3[...], 0.0)
    x16 = mm(h, bd23) + b2t3[...]

    gx = mm(x16, rmat)
    hg = jnp.maximum(mm(gx, g3w1) + g3b1[...], 0.0)
    xg = mm(hg, g3w2) + g3b2[...]

    qw, qx, qy, qz = xg[:, 0:1], xg[:, 1:2], xg[:, 2:3], xg[:, 3:4]
    qn = jnp.sqrt(qw * qw + qx * qx + qy * qy + qz * qz) + 1e-8
    qw, qx, qy, qz = qw / qn, qx / qn, qy / qn, qz / qn
    m00 = 1 - 2 * (qy * qy + qz * qz)
    m01 = 2 * (qx * qy - qw * qz)
    m02 = 2 * (qx * qz + qw * qy)
    m10 = 2 * (qx * qy + qw * qz)
    m11 = 1 - 2 * (qx * qx + qz * qz)
    m12 = 2 * (qy * qz - qw * qx)
    m20 = 2 * (qx * qz - qw * qy)
    m21 = 2 * (qy * qz + qw * qx)
    m22 = 1 - 2 * (qx * qx + qy * qy)
    rcx = m00 * cx + m01 * cy + m02 * cz
    rcy = m10 * cx + m11 * cy + m12 * cz
    rcz = m20 * cx + m21 * cy + m22 * cz
    rc = jnp.concatenate([rcx, rcy, rcz], axis=1)

    h = jnp.maximum(
        mm(x16, bd4a) + mm(xg[:, 4:12], tb4) + mm(rc, pc4) + b4t[...], 0.0
    )
    y = mm(h, bd4b) + b4b[...]

    ymax = jnp.max(y, axis=1, keepdims=True)
    ey = jnp.exp(y - ymax)
    out_ref[...] = ey / jnp.sum(ey, axis=1, keepdims=True)


def _make_consts(params):
    eye = jnp.eye(_K, dtype=jnp.float32)

    def bd(w):
        return jnp.kron(eye, w)

    def fold_first(w, fin):
        return jnp.einsum("fh,kK->fkKh", w, eye).reshape(fin * _K, _K * w.shape[1])

    def tile_b(b):
        return jnp.tile(b.reshape(1, -1), (1, _K))

    l1w1, l1b1, l1w2, l1b2 = params["layer1"]
    gw1, gb1, gw2, gb2 = params["layerg"]
    l2w1, l2b1, l2w2, l2b2 = params["layer2"]
    g2w1, g2b1, g2w2, g2b2 = params["layerg2"]
    l3w1, l3b1, l3w2, l3b2 = params["layer3"]
    g3w1, g3b1, g3w2, g3b2 = params["layerg3"]
    l4w1, l4b1, l4w2, l4b2 = params["layer4"]

    consts = [
        jnp.tile(eye, (_K, 1)) / _K,
        fold_first(l1w1, 8), tile_b(l1b1), bd(l1w2), tile_b(l1b2),
        gw1, gb1.reshape(1, -1), gw2, gb2.reshape(1, -1),
        bd(l2w1[:16]), jnp.tile(l2w1[16:24], (1, _K)), tile_b(l2b1),
        bd(l2w2), tile_b(l2b2),
        g2w1, g2b1.reshape(1, -1), g2w2, g2b2.reshape(1, -1),
        bd(l3w1[:16]), jnp.tile(l3w1[16:24], (1, _K)), tile_b(l3b1),
        bd(l3w2), tile_b(l3b2),
        g3w1, g3b1.reshape(1, -1), g3w2, g3b2.reshape(1, -1),
        bd(l4w1[:16]), jnp.tile(l4w1[16:24], (1, _K)), fold_first(l4w1[24:27], 3),
        tile_b(l4b1), bd(l4w2), tile_b(l4b2),
    ]
    return consts


def _tc_forward(std, pos, nrm, g6, w, consts):
    nblk = _N // _T
    in_specs = [
        pl.BlockSpec(memory_space=pltpu.SMEM),
        pl.BlockSpec((_T, 3), lambda i: (i, 0)),
        pl.BlockSpec((_T, 3), lambda i: (i, 0)),
    ] + [pl.BlockSpec((_T, _K), lambda i: (i, 0)) for _ in range(7)] + [
        pl.BlockSpec(c.shape, lambda i: tuple([0] * c.ndim)) for c in consts
    ]
    out = pl.pallas_call(
        _tc_body,
        grid=(nblk,),
        in_specs=in_specs,
        out_specs=pl.BlockSpec((_T, _K), lambda i: (i, 0)),
        out_shape=jax.ShapeDtypeStruct((_N, _K), jnp.float32),
    )(std, pos, nrm, *g6, w, *consts)
    return out


def kernel(pos, old_weights, normals, edge_index, dense_l, stddev, params):
    cols = edge_index[1]
    tables = [pos[:, 0], pos[:, 1], pos[:, 2],
              normals[:, 0], normals[:, 1], normals[:, 2]]
    g6 = [a.reshape(_N, _K) for a in _sc_gather6(tables, cols)]
    w = old_weights.reshape(_N, _K)
    std = stddev.reshape(1, 1)
    consts = _make_consts(params)
    return _tc_forward(std, pos, normals, g6, w, consts)


# speedup vs baseline: 17.6160x; 1.0242x over previous
import functools

import jax
import jax.numpy as jnp
from jax import lax
from jax.experimental import pallas as pl
from jax.experimental.pallas import tpu as pltpu
from jax.experimental.pallas import tpu_sc as plsc

_N = 50000
_K = 16
_E = _N * _K
_T = 400
_NC = 2
_NS = 16
_CH = 5000


def _sc_gather6(tables, cols):
    e = cols.shape[0]
    nw = _NC * _NS
    b_per_w = e // nw
    n_ch = b_per_w // _CH
    mesh = plsc.VectorSubcoreMesh(core_axis_name="c", subcore_axis_name="s")

    @functools.partial(
        pl.kernel,
        out_type=[jax.ShapeDtypeStruct((e,), jnp.float32) for _ in range(6)],
        mesh=mesh,
        scratch_types=[pltpu.VMEM((_CH,), jnp.int32)]
        + [pltpu.VMEM((_CH,), jnp.float32) for _ in range(6)]
        + [pltpu.SemaphoreType.DMA],
    )
    def gather_kernel(*refs):
        tbls = refs[0:6]
        idx_hbm = refs[6]
        outs = refs[7:13]
        idx_v = refs[13]
        vals = refs[14:20]
        sem = refs[20]
        wid = lax.axis_index("s") * _NC + lax.axis_index("c")
        base = wid * b_per_w

        @pl.loop(0, n_ch)
        def _(c):
            off = base + c * _CH
            pltpu.sync_copy(idx_hbm.at[pl.ds(off, _CH)], idx_v)
            copies = [
                pltpu.async_copy(tbls[j].at[idx_v], vals[j], sem)
                for j in range(6)
            ]
            for cp in copies:
                cp.wait()
            for j in range(6):
                pltpu.sync_copy(vals[j], outs[j].at[pl.ds(off, _CH)])

    return gather_kernel(*tables, cols)


def _angle16(c0, c1, c2, d):
    return jnp.arctan2(jnp.sqrt(c0 * c0 + c1 * c1 + c2 * c2), d)


def _tc_body(std_ref, pos_ref, nrm_ref, g0, g1, g2, g3, g4, g5, w_ref, *rest):
    (rmat,
     p1, b1t, bd12, b2t1,
     gw1, gb1, gw2, gb2,
     bda2, tb2, b1t2, bd22, b2t2,
     g2w1, g2b1, g2w2, g2b2,
     bda3, tb3, b1t3, bd23, b2t3,
     g3w1, g3b1, g3w2, g3b2,
     bd4a, tb4, pc4, b4t, bd4b, b4b,
     out_ref) = rest

    def mm(a, b):
        return jnp.dot(a, b[...], preferred_element_type=jnp.float32)

    s = 0.2 / std_ref[0, 0]
    prx, pry, prz = pos_ref[:, 0:1], pos_ref[:, 1:2], pos_ref[:, 2:3]
    nrx, nry, nrz = nrm_ref[:, 0:1], nrm_ref[:, 1:2], nrm_ref[:, 2:3]
    pcx, pcy, pcz = g0[...], g1[...], g2[...]
    ncx, ncy, ncz = g3[...], g4[...], g5[...]

    cx = (pcx - prx) * s
    cy = (pcy - pry) * s
    cz = (pcz - prz) * s
    cn = jnp.sqrt(cx * cx + cy * cy + cz * cz)
    a1 = _angle16(nry * cz - nrz * cy, nrz * cx - nrx * cz, nrx * cy - nry * cx,
                  nrx * cx + nry * cy + nrz * cz)
    a2 = _angle16(ncy * cz - ncz * cy, ncz * cx - ncx * cz, ncx * cy - ncy * cx,
                  ncx * cx + ncy * cy + ncz * cz)
    a3 = _angle16(nry * ncz - nrz * ncy, nrz * ncx - nrx * ncz,
                  nrx * ncy - nry * ncx,
                  nrx * ncx + nry * ncy + nrz * ncz)

    x128 = jnp.concatenate([cx, cy, cz, w_ref[...], cn, a1, a2, a3], axis=1)
    h = jnp.maximum(mm(x128, p1) + b1t[...], 0.0)
    x16 = mm(h, bd12) + b2t1[...]

    gx = mm(x16, rmat)
    gin = jnp.concatenate([gx, nrm_ref[...]], axis=1)
    hg = jnp.maximum(mm(gin, gw1) + gb1[...], 0.0)
    xg = mm(hg, gw2) + gb2[...]

    h = jnp.maximum(mm(x16, bda2) + mm(xg, tb2) + b1t2[...], 0.0)
    x16 = mm(h, bd22) + b2t2[...]

    gx = mm(x16, rmat)
    hg = jnp.maximum(mm(gx, g2w1) + g2b1[...], 0.0)
    xg = mm(hg, g2w2) + g2b2[...]

    h = jnp.maximum(mm(x16, bda3) + mm(xg, tb3) + b1t3[...], 0.0)
    x16 = mm(h, bd23) + b2t3[...]

    gx = mm(x16, rmat)
    hg = jnp.maximum(mm(gx, g3w1) + g3b1[...], 0.0)
    xg = mm(hg, g3w2) + g3b2[...]

    qw, qx, qy, qz = xg[:, 0:1], xg[:, 1:2], xg[:, 2:3], xg[:, 3:4]
    qn = jnp.sqrt(qw * qw + qx * qx + qy * qy + qz * qz) + 1e-8
    qw, qx, qy, qz = qw / qn, qx / qn, qy / qn, qz / qn
    m00 = 1 - 2 * (qy * qy + qz * qz)
    m01 = 2 * (qx * qy - qw * qz)
    m02 = 2 * (qx * qz + qw * qy)
    m10 = 2 * (qx * qy + qw * qz)
    m11 = 1 - 2 * (qx * qx + qz * qz)
    m12 = 2 * (qy * qz - qw * qx)
    m20 = 2 * (qx * qz - qw * qy)
    m21 = 2 * (qy * qz + qw * qx)
    m22 = 1 - 2 * (qx * qx + qy * qy)
    rcx = m00 * cx + m01 * cy + m02 * cz
    rcy = m10 * cx + m11 * cy + m12 * cz
    rcz = m20 * cx + m21 * cy + m22 * cz
    rc = jnp.concatenate([rcx, rcy, rcz], axis=1)

    h = jnp.maximum(
        mm(x16, bd4a) + mm(xg[:, 4:12], tb4) + mm(rc, pc4) + b4t[...], 0.0
    )
    y = mm(h, bd4b) + b4b[...]

    ymax = jnp.max(y, axis=1, keepdims=True)
    ey = jnp.exp(y - ymax)
    out_ref[...] = ey / jnp.sum(ey, axis=1, keepdims=True)


def _make_consts(params):
    eye = jnp.eye(_K, dtype=jnp.float32)

    def bd(w):
        return jnp.kron(eye, w)

    def fold_first(w, fin):
        return jnp.einsum("fh,kK->fkKh", w, eye).reshape(fin * _K, _K * w.shape[1])

    def tile_b(b):
        return jnp.tile(b.reshape(1, -1), (1, _K))

    l1w1, l1b1, l1w2, l1b2 = params["layer1"]
    gw1, gb1, gw2, gb2 = params["layerg"]
    l2w1, l2b1, l2w2, l2b2 = params["layer2"]
    g2w1, g2b1, g2w2, g2b2 = params["layerg2"]
    l3w1, l3b1, l3w2, l3b2 = params["layer3"]
    g3w1, g3b1, g3w2, g3b2 = params["layerg3"]
    l4w1, l4b1, l4w2, l4b2 = params["layer4"]

    consts = [
        jnp.tile(eye, (_K, 1)) / _K,
        fold_first(l1w1, 8), tile_b(l1b1), bd(l1w2), tile_b(l1b2),
        gw1, gb1.reshape(1, -1), gw2, gb2.reshape(1, -1),
        bd(l2w1[:16]), jnp.tile(l2w1[16:24], (1, _K)), tile_b(l2b1),
        bd(l2w2), tile_b(l2b2),
        g2w1, g2b1.reshape(1, -1), g2w2, g2b2.reshape(1, -1),
        bd(l3w1[:16]), jnp.tile(l3w1[16:24], (1, _K)), tile_b(l3b1),
        bd(l3w2), tile_b(l3b2),
        g3w1, g3b1.reshape(1, -1), g3w2, g3b2.reshape(1, -1),
        bd(l4w1[:16]), jnp.tile(l4w1[16:24], (1, _K)), fold_first(l4w1[24:27], 3),
        tile_b(l4b1), bd(l4w2), tile_b(l4b2),
    ]
    return consts


def _tc_forward(std, pos, nrm, g6, w, consts):
    n = pos.shape[0]
    nblk = n // _T
    in_specs = [
        pl.BlockSpec(memory_space=pltpu.SMEM),
        pl.BlockSpec((_T, 3), lambda i: (i, 0)),
        pl.BlockSpec((_T, 3), lambda i: (i, 0)),
    ] + [pl.BlockSpec((_T, _K), lambda i: (i, 0)) for _ in range(7)] + [
        pl.BlockSpec(c.shape, lambda i: tuple([0] * c.ndim)) for c in consts
    ]
    out = pl.pallas_call(
        _tc_body,
        grid=(nblk,),
        in_specs=in_specs,
        out_specs=pl.BlockSpec((_T, _K), lambda i: (i, 0)),
        out_shape=jax.ShapeDtypeStruct((n, _K), jnp.float32),
    )(std, pos, nrm, *g6, w, *consts)
    return out


_S = 5


def kernel(pos, old_weights, normals, edge_index, dense_l, stddev, params):
    cols = edge_index[1]
    tables = [pos[:, 0], pos[:, 1], pos[:, 2],
              normals[:, 0], normals[:, 1], normals[:, 2]]
    w = old_weights.reshape(_N, _K)
    std = stddev.reshape(1, 1)
    consts = _make_consts(params)

    nn = _N // _S
    ne = _E // _S
    gathered = [
        _sc_gather6(tables, lax.dynamic_slice_in_dim(cols, c * ne, ne))
        for c in range(_S)
    ]
    outs = []
    for c in range(_S):
        g6 = [a.reshape(nn, _K) for a in gathered[c]]
        outs.append(
            _tc_forward(
                std,
                lax.dynamic_slice_in_dim(pos, c * nn, nn),
                lax.dynamic_slice_in_dim(normals, c * nn, nn),
                g6,
                lax.dynamic_slice_in_dim(w, c * nn, nn),
                consts,
            )
        )
    return jnp.concatenate(outs, axis=0)


# speedup vs baseline: 19.3606x; 1.0990x over previous
import functools

import jax
import jax.numpy as jnp
from jax import lax
from jax.experimental import pallas as pl
from jax.experimental.pallas import tpu as pltpu
from jax.experimental.pallas import tpu_sc as plsc

_N = 50000
_K = 16
_E = _N * _K
_T = 400
_NC = 2
_NS = 16
_CH = 5000


def _sc_gather6(tables, cols):
    e = cols.shape[0]
    nw = _NC * _NS
    b_per_w = e // nw
    n_ch = b_per_w // _CH
    mesh = plsc.VectorSubcoreMesh(core_axis_name="c", subcore_axis_name="s")

    @functools.partial(
        pl.kernel,
        out_type=[jax.ShapeDtypeStruct((e,), jnp.float32) for _ in range(6)],
        mesh=mesh,
        scratch_types=[pltpu.VMEM((_CH,), jnp.int32)]
        + [pltpu.VMEM((_CH,), jnp.float32) for _ in range(6)]
        + [pltpu.SemaphoreType.DMA],
    )
    def gather_kernel(*refs):
        tbls = refs[0:6]
        idx_hbm = refs[6]
        outs = refs[7:13]
        idx_v = refs[13]
        vals = refs[14:20]
        sem = refs[20]
        wid = lax.axis_index("s") * _NC + lax.axis_index("c")
        base = wid * b_per_w

        @pl.loop(0, n_ch)
        def _(c):
            off = base + c * _CH
            pltpu.sync_copy(idx_hbm.at[pl.ds(off, _CH)], idx_v)
            copies = [
                pltpu.async_copy(tbls[j].at[idx_v], vals[j], sem)
                for j in range(6)
            ]
            for cp in copies:
                cp.wait()
            for j in range(6):
                pltpu.sync_copy(vals[j], outs[j].at[pl.ds(off, _CH)])

    return gather_kernel(*tables, cols)


def _tc_body(std_ref, pos_ref, nrm_ref, g0, g1, g2, g3, g4, g5, w_ref, *rest):
    (rmat,
     p1, b1t, bd12, b2t1,
     gw1, gb1, gw2, gb2,
     bda2, tb2, b1t2, bd22, b2t2,
     g2w1, g2b1, g2w2, g2b2,
     bda3, tb3, b1t3, bd23, b2t3,
     g3w1, g3b1, g3w2, g3b2,
     bd4a, tb4, pc4, b4t, bd4b, b4b,
     out_ref) = rest

    def mm(a, b):
        return jnp.dot(a, b[...], preferred_element_type=jnp.float32)

    s = 0.2 / std_ref[0, 0]
    prx, pry, prz = pos_ref[:, 0:1], pos_ref[:, 1:2], pos_ref[:, 2:3]
    nrx, nry, nrz = nrm_ref[:, 0:1], nrm_ref[:, 1:2], nrm_ref[:, 2:3]
    pcx, pcy, pcz = g0[...], g1[...], g2[...]
    ncx, ncy, ncz = g3[...], g4[...], g5[...]

    cx = (pcx - prx) * s
    cy = (pcy - pry) * s
    cz = (pcz - prz) * s

    def sqn(u0, u1, u2):
        return u0 * u0 + u1 * u1 + u2 * u2

    s1 = sqn(nry * cz - nrz * cy, nrz * cx - nrx * cz, nrx * cy - nry * cx)
    s2 = sqn(ncy * cz - ncz * cy, ncz * cx - ncx * cz, ncx * cy - ncy * cx)
    s3 = sqn(nry * ncz - nrz * ncy, nrz * ncx - nrx * ncz, nrx * ncy - nry * ncx)
    rt = jnp.sqrt(jnp.concatenate([s1, s2, s3, sqn(cx, cy, cz)], axis=1))
    dots = jnp.concatenate(
        [nrx * cx + nry * cy + nrz * cz,
         ncx * cx + ncy * cy + ncz * cz,
         nrx * ncx + nry * ncy + nrz * ncz], axis=1)
    ang = jnp.arctan2(rt[:, 0:48], dots)

    x128 = jnp.concatenate([cx, cy, cz, w_ref[...], rt[:, 48:64], ang], axis=1)
    h = jnp.maximum(mm(x128, p1) + b1t[...], 0.0)
    x16 = mm(h, bd12) + b2t1[...]

    gx = mm(x16, rmat)
    gin = jnp.concatenate([gx, nrm_ref[...]], axis=1)
    hg = jnp.maximum(mm(gin, gw1) + gb1[...], 0.0)
    xg = mm(hg, gw2) + gb2[...]

    h = jnp.maximum(mm(x16, bda2) + mm(xg, tb2) + b1t2[...], 0.0)
    x16 = mm(h, bd22) + b2t2[...]

    gx = mm(x16, rmat)
    hg = jnp.maximum(mm(gx, g2w1) + g2b1[...], 0.0)
    xg = mm(hg, g2w2) + g2b2[...]

    h = jnp.maximum(mm(x16, bda3) + mm(xg, tb3) + b1t3[...], 0.0)
    x16 = mm(h, bd23) + b2t3[...]

    gx = mm(x16, rmat)
    hg = jnp.maximum(mm(gx, g3w1) + g3b1[...], 0.0)
    xg = mm(hg, g3w2) + g3b2[...]

    qw, qx, qy, qz = xg[:, 0:1], xg[:, 1:2], xg[:, 2:3], xg[:, 3:4]
    d = qw * qw + qx * qx + qy * qy + qz * qz + 1e-16
    r = 1.0 / d
    m00 = d - 2 * (qy * qy + qz * qz)
    m01 = 2 * (qx * qy - qw * qz)
    m02 = 2 * (qx * qz + qw * qy)
    m10 = 2 * (qx * qy + qw * qz)
    m11 = d - 2 * (qx * qx + qz * qz)
    m12 = 2 * (qy * qz - qw * qx)
    m20 = 2 * (qx * qz - qw * qy)
    m21 = 2 * (qy * qz + qw * qx)
    m22 = d - 2 * (qx * qx + qy * qy)
    rcx = (m00 * cx + m01 * cy + m02 * cz) * r
    rcy = (m10 * cx + m11 * cy + m12 * cz) * r
    rcz = (m20 * cx + m21 * cy + m22 * cz) * r
    rc = jnp.concatenate([rcx, rcy, rcz], axis=1)

    h = jnp.maximum(
        mm(x16, bd4a) + mm(xg[:, 4:12], tb4) + mm(rc, pc4) + b4t[...], 0.0
    )
    y = mm(h, bd4b) + b4b[...]

    ymax = jnp.max(y, axis=1, keepdims=True)
    ey = jnp.exp(y - ymax)
    out_ref[...] = ey / jnp.sum(ey, axis=1, keepdims=True)


def _make_consts(params):
    eye = jnp.eye(_K, dtype=jnp.float32)

    def bd(w):
        return jnp.kron(eye, w)

    def fold_first(w, fin):
        return jnp.einsum("fh,kK->fkKh", w, eye).reshape(fin * _K, _K * w.shape[1])

    def tile_b(b):
        return jnp.tile(b.reshape(1, -1), (1, _K))

    l1w1, l1b1, l1w2, l1b2 = params["layer1"]
    gw1, gb1, gw2, gb2 = params["layerg"]
    l2w1, l2b1, l2w2, l2b2 = params["layer2"]
    g2w1, g2b1, g2w2, g2b2 = params["layerg2"]
    l3w1, l3b1, l3w2, l3b2 = params["layer3"]
    g3w1, g3b1, g3w2, g3b2 = params["layerg3"]
    l4w1, l4b1, l4w2, l4b2 = params["layer4"]

    consts = [
        jnp.tile(eye, (_K, 1)) / _K,
        fold_first(l1w1, 8), tile_b(l1b1), bd(l1w2), tile_b(l1b2),
        gw1, gb1.reshape(1, -1), gw2, gb2.reshape(1, -1),
        bd(l2w1[:16]), jnp.tile(l2w1[16:24], (1, _K)), tile_b(l2b1),
        bd(l2w2), tile_b(l2b2),
        g2w1, g2b1.reshape(1, -1), g2w2, g2b2.reshape(1, -1),
        bd(l3w1[:16]), jnp.tile(l3w1[16:24], (1, _K)), tile_b(l3b1),
        bd(l3w2), tile_b(l3b2),
        g3w1, g3b1.reshape(1, -1), g3w2, g3b2.reshape(1, -1),
        bd(l4w1[:16]), jnp.tile(l4w1[16:24], (1, _K)), fold_first(l4w1[24:27], 3),
        tile_b(l4b1), bd(l4w2), tile_b(l4b2),
    ]
    return consts


def _tc_forward(std, pos, nrm, g6, w, consts):
    n = pos.shape[0]
    nblk = n // _T
    in_specs = [
        pl.BlockSpec(memory_space=pltpu.SMEM),
        pl.BlockSpec((_T, 3), lambda i: (i, 0)),
        pl.BlockSpec((_T, 3), lambda i: (i, 0)),
    ] + [pl.BlockSpec((_T, _K), lambda i: (i, 0)) for _ in range(7)] + [
        pl.BlockSpec(c.shape, lambda i: tuple([0] * c.ndim)) for c in consts
    ]
    out = pl.pallas_call(
        _tc_body,
        grid=(nblk,),
        in_specs=in_specs,
        out_specs=pl.BlockSpec((_T, _K), lambda i: (i, 0)),
        out_shape=jax.ShapeDtypeStruct((n, _K), jnp.float32),
        compiler_params=pltpu.CompilerParams(
            dimension_semantics=("parallel",)
        ),
    )(std, pos, nrm, *g6, w, *consts)
    return out


_S = 5


def kernel(pos, old_weights, normals, edge_index, dense_l, stddev, params):
    cols = edge_index[1]
    tables = [pos[:, 0], pos[:, 1], pos[:, 2],
              normals[:, 0], normals[:, 1], normals[:, 2]]
    w = old_weights.reshape(_N, _K)
    std = stddev.reshape(1, 1)
    consts = _make_consts(params)

    nn = _N // _S
    ne = _E // _S
    gathered = [
        _sc_gather6(tables, lax.dynamic_slice_in_dim(cols, c * ne, ne))
        for c in range(_S)
    ]
    outs = []
    for c in range(_S):
        g6 = [a.reshape(nn, _K) for a in gathered[c]]
        outs.append(
            _tc_forward(
                std,
                lax.dynamic_slice_in_dim(pos, c * nn, nn),
                lax.dynamic_slice_in_dim(normals, c * nn, nn),
                g6,
                lax.dynamic_slice_in_dim(w, c * nn, nn),
                consts,
            )
        )
    return jnp.concatenate(outs, axis=0)
